# baseline (speedup 1.0000x reference)
import functools

import jax
import jax.numpy as jnp
from jax import lax
from jax.experimental import pallas as pl
from jax.experimental.pallas import tpu as pltpu
from jax.experimental.pallas import tpu_sc as plsc

_B, _D, _E, _C = 4096, 1024, 8, 1024
_T = 256
_NP = _B + _E * _T
_NT = _NP // _T
_NW = 32
_RPW = _B // _NW
_CH = 32
_NCH = _RPW // _CH

_sc_mesh = plsc.VectorSubcoreMesh(core_axis_name="c", subcore_axis_name="s")

_GUMBEL_CACHE = []


def _gumbel_noise():
    if not _GUMBEL_CACHE:
        _GUMBEL_CACHE.append(
            jax.random.gumbel(jax.random.key(42), (_B, _E), jnp.float32)
        )
    return _GUMBEL_CACHE[0]


_SC_SCRATCH = [
    pltpu.VMEM((_NCH, _CH), jnp.int32),
    pltpu.VMEM((_CH, _D), jnp.float32),
    pltpu.VMEM((_CH, _D), jnp.float32),
    pltpu.SemaphoreType.DMA,
    pltpu.SemaphoreType.DMA,
    pltpu.SemaphoreType.DMA,
]


@functools.partial(
    pl.kernel,
    out_type=jax.ShapeDtypeStruct((_NP, _D), jnp.float32),
    mesh=_sc_mesh,
    scratch_types=_SC_SCRATCH,
)
def _sc_scatter_rows(x_hbm, dest2_hbm, xs_hbm, idx_v, bufa, bufb, sema, semb, sems):
    wid = lax.axis_index("s") * 2 + lax.axis_index("c")
    base = wid * _RPW
    pltpu.sync_copy(dest2_hbm.at[pl.ds(wid * _NCH, _NCH)], idx_v)
    bufs, lsems = (bufa, bufb), (sema, semb)
    loads = [None] * _NCH
    loads[0] = pltpu.async_copy(x_hbm.at[pl.ds(base, _CH)], bufa, sema)
    for c in range(_NCH):
        if c + 1 < _NCH:
            loads[c + 1] = pltpu.async_copy(
                x_hbm.at[pl.ds(base + (c + 1) * _CH, _CH)],
                bufs[(c + 1) % 2],
                lsems[(c + 1) % 2],
            )
        loads[c].wait()
        pltpu.async_copy(bufs[c % 2], xs_hbm.at[idx_v.at[c]], sems).wait()


@functools.partial(
    pl.kernel,
    out_type=jax.ShapeDtypeStruct((_B, _C), jnp.float32),
    mesh=_sc_mesh,
    scratch_types=_SC_SCRATCH,
)
def _sc_gather_rows(ys_hbm, dest2_hbm, out_hbm, idx_v, bufa, bufb, sema, semb, sems):
    wid = lax.axis_index("s") * 2 + lax.axis_index("c")
    base = wid * _RPW
    pltpu.sync_copy(dest2_hbm.at[pl.ds(wid * _NCH, _NCH)], idx_v)
    bufs, gsems = (bufa, bufb), (sema, semb)
    gets = [None] * _NCH
    gets[0] = pltpu.async_copy(ys_hbm.at[idx_v.at[0]], bufa, sema)
    for c in range(_NCH):
        if c + 1 < _NCH:
            gets[c + 1] = pltpu.async_copy(
                ys_hbm.at[idx_v.at[c + 1]],
                bufs[(c + 1) % 2],
                gsems[(c + 1) % 2],
            )
        gets[c].wait()
        pltpu.async_copy(
            bufs[c % 2], out_hbm.at[pl.ds(base + c * _CH, _CH)], sems
        ).wait()


def _route_body(s_ref, dest_ref, te_ref):
    s = s_ref[...]
    triu = (
        lax.broadcasted_iota(jnp.int32, (128, 128), 0)
        <= lax.broadcasted_iota(jnp.int32, (128, 128), 1)
    ).astype(jnp.float32)
    lstrict = (
        lax.broadcasted_iota(jnp.int32, (32, 32), 1)
        < lax.broadcasted_iota(jnp.int32, (32, 32), 0)
    ).astype(jnp.float32)
    rs_cols = [
        jnp.sum((s == e).astype(jnp.float32), axis=1, keepdims=True)
        for e in range(_E)
    ]
    rs = jnp.concatenate(rs_cols, axis=1)
    pref = jnp.dot(lstrict, rs, preferred_element_type=jnp.float32)
    counts = jnp.sum(rs, axis=0, keepdims=True)
    cap = jnp.floor((counts + float(_T - 1)) / float(_T)) * float(_T)
    ends_cols = []
    run = jnp.zeros((1, 1), jnp.float32)
    for e in range(_E):
        run = run + cap[:, e : e + 1]
        ends_cols.append(run)
    ends = jnp.concatenate(ends_cols, axis=1)
    ao = ends - cap
    dest = jnp.zeros((32, 128), jnp.float32)
    for e in range(_E):
        ohe = (s == e).astype(jnp.float32)
        incl = jnp.dot(ohe, triu, preferred_element_type=jnp.float32)
        ranke = pref[:, e : e + 1] + incl - ohe
        dest = dest + ohe * (ao[:, e : e + 1] + ranke)
    dest_ref[...] = dest.astype(jnp.int32)
    tstart = lax.broadcasted_iota(jnp.int32, (1, _NT), 1).astype(
        jnp.float32
    ) * float(_T)
    acc = jnp.zeros((1, _NT), jnp.float32)
    for e in range(_E):
        acc = acc + (tstart >= ends[:, e : e + 1]).astype(jnp.float32)
    te_ref[...] = jnp.minimum(acc, float(_E - 1)).astype(jnp.int32)


def _route(sample):
    dest2, te2 = pl.pallas_call(
        _route_body,
        out_shape=(
            jax.ShapeDtypeStruct((32, 128), jnp.int32),
            jax.ShapeDtypeStruct((1, _NT), jnp.int32),
        ),
    )(sample.reshape(32, 128))
    return dest2.reshape(_B), te2.reshape(_NT)


_TPS = 6
_BT = _T * _TPS


def _mm_body(te_ref, x_ref, w_ref, b_ref, o_ref):
    i = pl.program_id(0)
    for k in range(_TPS):
        e = te_ref[i * _TPS + k]
        acc = jnp.dot(
            x_ref[k * _T : (k + 1) * _T, :],
            w_ref[e],
            preferred_element_type=jnp.float32,
        )
        o_ref[k * _T : (k + 1) * _T, :] = acc + b_ref[e]


def _expert_matmul(tile_expert, xs, expert_W, expert_b):
    grid_spec = pltpu.PrefetchScalarGridSpec(
        num_scalar_prefetch=1,
        grid=(_NT // _TPS,),
        in_specs=[
            pl.BlockSpec((_BT, _D), lambda i, te: (i, 0)),
            pl.BlockSpec((_E, _D, _C), lambda i, te: (0, 0, 0)),
            pl.BlockSpec((_E, 1, _C), lambda i, te: (0, 0, 0)),
        ],
        out_specs=pl.BlockSpec((_BT, _C), lambda i, te: (i, 0)),
    )
    return pl.pallas_call(
        _mm_body,
        grid_spec=grid_spec,
        out_shape=jax.ShapeDtypeStruct((_NP, _C), jnp.float32),
    )(tile_expert, xs, expert_W, expert_b.reshape(_E, 1, _C))


def kernel(inputs, expert_W, expert_b, gate_W, gate_b):
    logits = inputs @ gate_W + gate_b
    p = jax.nn.softmax(logits, axis=-1)
    sample = jnp.argmax(jnp.log(p) + _gumbel_noise(), axis=-1)
    sample = sample.astype(jnp.int32)

    dest, tile_expert = _route(sample)

    dest2 = dest.reshape(_NW * _NCH, _CH)
    xs = _sc_scatter_rows(inputs, dest2)
    ys = _expert_matmul(tile_expert, xs, expert_W, expert_b)
    return _sc_gather_rows(ys, dest2)


# speedup vs baseline: 1.0390x; 1.0390x over previous
import functools

import jax
import jax.numpy as jnp
from jax import lax
from jax.experimental import pallas as pl
from jax.experimental.pallas import tpu as pltpu
from jax.experimental.pallas import tpu_sc as plsc

_B, _D, _E, _C = 4096, 1024, 8, 1024
_T = 128
_NP = _B + _E * _T
_NT = _NP // _T
_NW = 32
_RPW = _B // _NW
_CH = 64
_NCH = _RPW // _CH

_sc_mesh = plsc.VectorSubcoreMesh(core_axis_name="c", subcore_axis_name="s")

_GUMBEL_CACHE = []


def _gumbel_noise():
    if not _GUMBEL_CACHE:
        _GUMBEL_CACHE.append(
            jax.random.gumbel(jax.random.key(42), (_B, _E), jnp.float32)
        )
    return _GUMBEL_CACHE[0]


_SC_SCRATCH = [
    pltpu.VMEM((_NCH, _CH), jnp.int32),
    pltpu.VMEM((_CH, _D), jnp.float32),
    pltpu.SemaphoreType.DMA,
]


@functools.partial(
    pl.kernel,
    out_type=jax.ShapeDtypeStruct((_NP, _D), jnp.float32),
    mesh=_sc_mesh,
    scratch_types=_SC_SCRATCH,
)
def _sc_scatter_rows(x_hbm, dest2_hbm, xs_hbm, idx_v, rows_v, sem):
    wid = lax.axis_index("s") * 2 + lax.axis_index("c")
    base = wid * _RPW
    pltpu.sync_copy(dest2_hbm.at[pl.ds(wid * _NCH, _NCH)], idx_v)
    for c in range(_NCH):
        pltpu.sync_copy(x_hbm.at[pl.ds(base + c * _CH, _CH)], rows_v)
        pltpu.async_copy(rows_v, xs_hbm.at[idx_v.at[c]], sem).wait()


@functools.partial(
    pl.kernel,
    out_type=jax.ShapeDtypeStruct((_B, _C), jnp.float32),
    mesh=_sc_mesh,
    scratch_types=_SC_SCRATCH,
)
def _sc_gather_rows(ys_hbm, dest2_hbm, out_hbm, idx_v, rows_v, sem):
    wid = lax.axis_index("s") * 2 + lax.axis_index("c")
    base = wid * _RPW
    pltpu.sync_copy(dest2_hbm.at[pl.ds(wid * _NCH, _NCH)], idx_v)
    for c in range(_NCH):
        pltpu.async_copy(ys_hbm.at[idx_v.at[c]], rows_v, sem).wait()
        pltpu.sync_copy(rows_v, out_hbm.at[pl.ds(base + c * _CH, _CH)])


def _route_body(s_ref, dest_ref, te_ref):
    s = s_ref[...]
    triu = (
        lax.broadcasted_iota(jnp.int32, (128, 128), 0)
        <= lax.broadcasted_iota(jnp.int32, (128, 128), 1)
    ).astype(jnp.float32)
    lstrict = (
        lax.broadcasted_iota(jnp.int32, (32, 32), 1)
        < lax.broadcasted_iota(jnp.int32, (32, 32), 0)
    ).astype(jnp.float32)
    rs_cols = [
        jnp.sum((s == e).astype(jnp.float32), axis=1, keepdims=True)
        for e in range(_E)
    ]
    rs = jnp.concatenate(rs_cols, axis=1)
    pref = jnp.dot(lstrict, rs, preferred_element_type=jnp.float32)
    counts = jnp.sum(rs, axis=0, keepdims=True)
    cap = jnp.floor((counts + float(_T - 1)) / float(_T)) * float(_T)
    ends_cols = []
    run = jnp.zeros((1, 1), jnp.float32)
    for e in range(_E):
        run = run + cap[:, e : e + 1]
        ends_cols.append(run)
    ends = jnp.concatenate(ends_cols, axis=1)
    ao = ends - cap
    dest = jnp.zeros((32, 128), jnp.float32)
    for e in range(_E):
        ohe = (s == e).astype(jnp.float32)
        incl = jnp.dot(ohe, triu, preferred_element_type=jnp.float32)
        ranke = pref[:, e : e + 1] + incl - ohe
        dest = dest + ohe * (ao[:, e : e + 1] + ranke)
    dest_ref[...] = dest.astype(jnp.int32)
    tstart = lax.broadcasted_iota(jnp.int32, (1, _NT), 1).astype(
        jnp.float32
    ) * float(_T)
    acc = jnp.zeros((1, _NT), jnp.float32)
    for e in range(_E):
        acc = acc + (tstart >= ends[:, e : e + 1]).astype(jnp.float32)
    te_ref[...] = jnp.minimum(acc, float(_E - 1)).astype(jnp.int32)


def _route(sample):
    dest2, te2 = pl.pallas_call(
        _route_body,
        out_shape=(
            jax.ShapeDtypeStruct((32, 128), jnp.int32),
            jax.ShapeDtypeStruct((1, _NT), jnp.int32),
        ),
    )(sample.reshape(32, 128))
    return dest2.reshape(_B), te2.reshape(_NT)


_TPS = 8
_BT = _T * _TPS


def _mm_body(te_ref, x_ref, w_ref, b_ref, o_ref):
    i = pl.program_id(0)
    for k in range(_TPS):
        e = te_ref[i * _TPS + k]
        acc = jnp.dot(
            x_ref[k * _T : (k + 1) * _T, :],
            w_ref[e],
            preferred_element_type=jnp.float32,
        )
        o_ref[k * _T : (k + 1) * _T, :] = acc + b_ref[e]


def _expert_matmul(tile_expert, xs, expert_W, expert_b):
    grid_spec = pltpu.PrefetchScalarGridSpec(
        num_scalar_prefetch=1,
        grid=(_NT // _TPS,),
        in_specs=[
            pl.BlockSpec((_BT, _D), lambda i, te: (i, 0)),
            pl.BlockSpec((_E, _D, _C), lambda i, te: (0, 0, 0)),
            pl.BlockSpec((_E, 1, _C), lambda i, te: (0, 0, 0)),
        ],
        out_specs=pl.BlockSpec((_BT, _C), lambda i, te: (i, 0)),
    )
    return pl.pallas_call(
        _mm_body,
        grid_spec=grid_spec,
        out_shape=jax.ShapeDtypeStruct((_NP, _C), jnp.float32),
    )(tile_expert, xs, expert_W, expert_b.reshape(_E, 1, _C))


def kernel(inputs, expert_W, expert_b, gate_W, gate_b):
    logits = inputs @ gate_W + gate_b
    p = jax.nn.softmax(logits, axis=-1)
    sample = jnp.argmax(jnp.log(p) + _gumbel_noise(), axis=-1)
    sample = sample.astype(jnp.int32)

    dest, tile_expert = _route(sample)

    dest2 = dest.reshape(_NW * _NCH, _CH)
    xs = _sc_scatter_rows(inputs, dest2)
    ys = _expert_matmul(tile_expert, xs, expert_W, expert_b)
    return _sc_gather_rows(ys, dest2)
